# Initial kernel scaffold; baseline (speedup 1.0000x reference)
import jax
import jax.numpy as jnp
from jax.experimental import pallas as pl


def kernel(feat, edge_index, labels, W1, b1, W2, b2):
    raise NotImplementedError("write your pallas kernel here")


# speedup vs baseline: 7.8457x; 7.8457x over previous
import dataclasses
import functools

import jax
import jax.numpy as jnp
from jax import lax
from jax.experimental import pallas as pl
from jax.experimental.pallas import tpu as pltpu
from jax.experimental.pallas import tpu_sc as plsc

N = 10000
E = 320000
D = 128
B = 8192
NC = 2
NS = 16
LANES = 16
NT = NC * NS

NPAD = 10240
NODES_PER_TILE = NPAD // NS
ACC_ROWS_PER_TILE = NPAD // NS
IDX_PER_TILE = E // NS

EDGE_CHUNK = 80
CHUNKS_PER_TILE = E // (EDGE_CHUNK * NT)

LAB_CHUNK = 128
LAB_CHUNKS_PER_TILE = B // (LAB_CHUNK * NT)

_mesh = plsc.VectorSubcoreMesh(core_axis_name="c", subcore_axis_name="s")

_sc_params = pltpu.CompilerParams()
if "needs_layout_passes" in pltpu.CompilerParams.__dataclass_fields__:
    _sc_params = dataclasses.replace(_sc_params, needs_layout_passes=False)


def _newton_rsqrt(m):
    xi = plsc.bitcast(m, jnp.int32)
    yi = jnp.int32(0x5F3759DF) - lax.shift_right_logical(xi, 1)
    y = plsc.bitcast(yi, jnp.float32)
    for _ in range(3):
        y = y * (1.5 - 0.5 * m * y * y)
    return y


@jax.jit
def _degree_norms(src, dst):
    @functools.partial(
        pl.kernel,
        out_type=(
            jax.ShapeDtypeStruct((NPAD,), jnp.float32),
            jax.ShapeDtypeStruct((NPAD,), jnp.float32),
        ),
        mesh=_mesh,
        compiler_params=_sc_params,
        scratch_types=[
            pltpu.VMEM((IDX_PER_TILE,), jnp.int32),
            pltpu.VMEM((NPAD,), jnp.float32),
            pltpu.VMEM((NS, NODES_PER_TILE), jnp.float32),
            pltpu.VMEM((NODES_PER_TILE,), jnp.float32),
            pltpu.VMEM_SHARED((NS, NPAD), jnp.float32),
        ],
    )
    def k(s_hbm, d_hbm, ons_hbm, ond_hbm, idx_v, hist_v, merge_v, norm_v, slab):
        c = lax.axis_index("c")
        s = lax.axis_index("s")

        @pl.when(c == 0)
        def _():
            pltpu.sync_copy(s_hbm.at[pl.ds(s * IDX_PER_TILE, IDX_PER_TILE)], idx_v)

        @pl.when(c == 1)
        def _():
            pltpu.sync_copy(d_hbm.at[pl.ds(s * IDX_PER_TILE, IDX_PER_TILE)], idx_v)

        zeros16 = jnp.zeros((LANES,), jnp.float32)

        @pl.loop(0, NPAD // LANES)
        def _(i):
            hist_v[pl.ds(i * LANES, LANES)] = zeros16

        ones16 = jnp.ones((LANES,), jnp.float32)

        @pl.loop(0, IDX_PER_TILE // LANES)
        def _(i):
            idx = idx_v[pl.ds(i * LANES, LANES)]
            plsc.addupdate_scatter(hist_v, [idx], ones16)

        pltpu.sync_copy(hist_v, slab.at[s])
        plsc.subcore_barrier()
        for t in range(NS):
            pltpu.sync_copy(
                slab.at[t, pl.ds(s * NODES_PER_TILE, NODES_PER_TILE)],
                merge_v.at[t],
            )

        @pl.loop(0, NODES_PER_TILE // LANES)
        def _(j):
            d = merge_v[0, pl.ds(j * LANES, LANES)]
            for t in range(1, NS):
                d = d + merge_v[t, pl.ds(j * LANES, LANES)]
            m = jnp.maximum(d, 1.0)
            r = _newton_rsqrt(m)
            norm_v[pl.ds(j * LANES, LANES)] = jnp.where(d > 0.0, r, 0.0)

        @pl.when(c == 0)
        def _():
            pltpu.sync_copy(
                norm_v, ons_hbm.at[pl.ds(s * NODES_PER_TILE, NODES_PER_TILE)]
            )

        @pl.when(c == 1)
        def _():
            pltpu.sync_copy(
                norm_v, ond_hbm.at[pl.ds(s * NODES_PER_TILE, NODES_PER_TILE)]
            )

    return k(src, dst)


@jax.jit
def _aggregate(h, srcr, dstr):
    @functools.partial(
        pl.kernel,
        out_type=jax.ShapeDtypeStruct((NC, NPAD, D), jnp.float32),
        mesh=_mesh,
        compiler_params=_sc_params,
        scratch_types=[
            pltpu.VMEM((CHUNKS_PER_TILE, EDGE_CHUNK), jnp.int32),
            pltpu.VMEM((CHUNKS_PER_TILE, EDGE_CHUNK), jnp.int32),
            pltpu.VMEM((EDGE_CHUNK, D), jnp.float32),
            pltpu.VMEM((8, D), jnp.float32),
            pltpu.VMEM_SHARED((NPAD, D), jnp.float32),
            pltpu.SemaphoreType.DMA,
        ],
    )
    def k(h_hbm, s_hbm, d_hbm, out_hbm, sidx, didx, rows, zbuf, acc, sem):
        c = lax.axis_index("c")
        s = lax.axis_index("s")
        t = c * NS + s

        zeros16 = jnp.zeros((LANES,), jnp.float32)

        @pl.loop(0, 8)
        def _(i):
            for j in range(D // LANES):
                zbuf[i, pl.ds(j * LANES, LANES)] = zeros16

        @pl.loop(0, ACC_ROWS_PER_TILE // 8)
        def _(q):
            pltpu.sync_copy(
                zbuf,
                acc.at[pl.ds(s * ACC_ROWS_PER_TILE + q * 8, 8)],
            )

        pltpu.sync_copy(s_hbm.at[t], sidx)
        pltpu.sync_copy(d_hbm.at[t], didx)
        plsc.subcore_barrier()

        @pl.loop(0, CHUNKS_PER_TILE)
        def _(i):
            pltpu.async_copy(h_hbm.at[sidx.at[i]], rows, sem).wait()
            pltpu.sync_copy(rows, acc.at[didx.at[i]], add=True)

        plsc.subcore_barrier()
        pltpu.sync_copy(
            acc.at[pl.ds(s * ACC_ROWS_PER_TILE, ACC_ROWS_PER_TILE)],
            out_hbm.at[c, pl.ds(s * ACC_ROWS_PER_TILE, ACC_ROWS_PER_TILE)],
        )

    return k(h, srcr, dstr)


@jax.jit
def _label_gather(h2, labr):
    @functools.partial(
        pl.kernel,
        out_type=jax.ShapeDtypeStruct((B, D), jnp.float32),
        mesh=_mesh,
        compiler_params=_sc_params,
        scratch_types=[
            pltpu.VMEM((LAB_CHUNKS_PER_TILE, LAB_CHUNK), jnp.int32),
            pltpu.VMEM((LAB_CHUNK, D), jnp.float32),
            pltpu.SemaphoreType.DMA,
        ],
    )
    def k(h_hbm, l_hbm, out_hbm, lab_v, rows_v, sem):
        c = lax.axis_index("c")
        s = lax.axis_index("s")
        t = c * NS + s
        pltpu.sync_copy(l_hbm.at[t], lab_v)
        for j in range(LAB_CHUNKS_PER_TILE):
            pltpu.async_copy(h_hbm.at[lab_v.at[j]], rows_v, sem).wait()
            pltpu.sync_copy(
                rows_v,
                out_hbm.at[
                    pl.ds((t * LAB_CHUNKS_PER_TILE + j) * LAB_CHUNK, LAB_CHUNK)
                ],
            )

    return k(h2, labr)


def _scale_body(f_ref, n_ref, o_ref):
    o_ref[...] = f_ref[...] * n_ref[...]


@jax.jit
def _scale(feat, ns_col):
    blk = 1000
    return pl.pallas_call(
        _scale_body,
        grid=(N // blk,),
        in_specs=[
            pl.BlockSpec((blk, D), lambda i: (i, 0)),
            pl.BlockSpec((blk, 1), lambda i: (i, 0)),
        ],
        out_specs=pl.BlockSpec((blk, D), lambda i: (i, 0)),
        out_shape=jax.ShapeDtypeStruct((N, D), jnp.float32),
    )(feat, ns_col)


def _mid_body(p_ref, nd_ref, ns_ref, w_ref, b_ref, o_ref):
    agg = p_ref[0] + p_ref[1]
    x = agg * nd_ref[...]
    y = jnp.dot(x, w_ref[...], preferred_element_type=jnp.float32) + b_ref[...]
    o_ref[...] = jnp.maximum(y, 0.0) * ns_ref[...]


@jax.jit
def _layer_mid(p, nd_col, ns_col, W, b_row):
    blk = 1024
    return pl.pallas_call(
        _mid_body,
        grid=(NPAD // blk,),
        in_specs=[
            pl.BlockSpec((NC, blk, D), lambda i: (0, i, 0)),
            pl.BlockSpec((blk, 1), lambda i: (i, 0)),
            pl.BlockSpec((blk, 1), lambda i: (i, 0)),
            pl.BlockSpec((D, D), lambda i: (0, 0)),
            pl.BlockSpec((1, D), lambda i: (0, 0)),
        ],
        out_specs=pl.BlockSpec((blk, D), lambda i: (i, 0)),
        out_shape=jax.ShapeDtypeStruct((NPAD, D), jnp.float32),
    )(p, nd_col, ns_col, W, b_row)


def _out_body(p_ref, nd_ref, w_ref, b_ref, o_ref):
    agg = p_ref[0] + p_ref[1]
    x = agg * nd_ref[...]
    o_ref[...] = (
        jnp.dot(x, w_ref[...], preferred_element_type=jnp.float32) + b_ref[...]
    )


@jax.jit
def _layer_out(p, nd_col, W, b_row):
    blk = 1024
    return pl.pallas_call(
        _out_body,
        grid=(NPAD // blk,),
        in_specs=[
            pl.BlockSpec((NC, blk, D), lambda i: (0, i, 0)),
            pl.BlockSpec((blk, 1), lambda i: (i, 0)),
            pl.BlockSpec((D, D), lambda i: (0, 0)),
            pl.BlockSpec((1, D), lambda i: (0, 0)),
        ],
        out_specs=pl.BlockSpec((blk, D), lambda i: (i, 0)),
        out_shape=jax.ShapeDtypeStruct((NPAD, D), jnp.float32),
    )(p, nd_col, W, b_row)


@jax.jit
def kernel(feat, edge_index, labels, W1, b1, W2, b2):
    ei = edge_index.astype(jnp.int32)
    src = ei[0]
    dst = ei[1]
    srcr = src.reshape(NT, CHUNKS_PER_TILE, EDGE_CHUNK)
    dstr = dst.reshape(NT, CHUNKS_PER_TILE, EDGE_CHUNK)
    labr = labels.astype(jnp.int32).reshape(NT, LAB_CHUNKS_PER_TILE, LAB_CHUNK)

    norm_s, norm_d = _degree_norms(src, dst)
    ns_col = norm_s.reshape(NPAD, 1)
    nd_col = norm_d.reshape(NPAD, 1)

    h1s = _scale(feat, ns_col[:N])
    p1 = _aggregate(h1s, srcr, dstr)
    h2s = _layer_mid(p1, nd_col, ns_col, W1, b1.reshape(1, D))
    p2 = _aggregate(h2s, srcr, dstr)
    h2 = _layer_out(p2, nd_col, W2, b2.reshape(1, D))
    return _label_gather(h2, labr)
